# baseline (speedup 1.0000x reference)
import functools

import jax
import jax.numpy as jnp
from jax import lax
from jax.experimental import pallas as pl
from jax.experimental.pallas import tpu as pltpu
from jax.experimental.pallas import tpu_sc as plsc

NUM_CORES = 2
NUM_SUBCORES = 16
NUM_WORKERS = NUM_CORES * NUM_SUBCORES
B = 128
ROW = 1024
ROWS_PER_BATCH = 256
BATCHES_PER_WORKER = B // NUM_WORKERS
CHUNK_ROWS = 32
CHUNK = CHUNK_ROWS * ROW
CHUNKS_PER_BATCH = ROWS_PER_BATCH // CHUNK_ROWS
TOTAL_CHUNKS = BATCHES_PER_WORKER * CHUNKS_PER_BATCH
PAIRS = TOTAL_CHUNKS // 2
HIST1 = 256 * 16
HIST = BATCHES_PER_WORKER * HIST1


def _make_kernel():
    mesh = plsc.VectorSubcoreMesh(
        core_axis_name="c", subcore_axis_name="s", num_cores=NUM_CORES
    )

    @functools.partial(
        pl.kernel,
        mesh=mesh,
        out_type=jax.ShapeDtypeStruct((B, 16), jnp.float32),
        compiler_params=pltpu.CompilerParams(
            needs_layout_passes=False,
            disable_bounds_checks=True,
            disable_semaphore_checks=True,
            skip_device_barrier=True,
        ),
        scratch_types=[
            pltpu.VMEM((CHUNK,), jnp.int32),
            pltpu.VMEM((CHUNK,), jnp.int32),
            pltpu.VMEM((HIST,), jnp.float32),
            pltpu.VMEM((16,), jnp.float32),
            pltpu.SemaphoreType.DMA,
            pltpu.SemaphoreType.DMA,
        ],
    )
    def hist_kernel(x_hbm, out_hbm, buf0, buf1, hist, res, sem0, sem1):
        w = lax.axis_index("s") * NUM_CORES + lax.axis_index("c")
        lane = lax.iota(jnp.int32, 16)
        ones = jnp.ones((16,), jnp.float32)
        fzero = jnp.zeros((16,), jnp.float32)
        word0 = w * BATCHES_PER_WORKER * ROWS_PER_BATCH * ROW

        def start(t, buf, sem):
            pltpu.async_copy(
                x_hbm.at[pl.ds(word0 + t * CHUNK, CHUNK)], buf, sem
            )

        def wait(buf, sem):
            pltpu.make_async_copy(
                x_hbm.at[pl.ds(0, CHUNK)], buf, sem
            ).wait()

        start(0, buf0, sem0)
        start(1, buf1, sem1)

        @plsc.parallel_loop(0, HIST, 16, unroll=8)
        def _(i):
            hist[pl.ds(i, 16)] = fzero

        def consume(t, buf, sem, last):
            wait(buf, sem)
            base = lane + (t >> 3) * HIST1

            @plsc.parallel_loop(0, CHUNK, 16, unroll=8)
            def _(i):
                vals = buf[pl.ds(i, 16)]
                idx = (vals << 4) + base
                plsc.addupdate_scatter(hist, [idx], ones)

            @pl.when(jnp.logical_not(last))
            def _():
                start(t + 2, buf, sem)

        def pbody(p, carry):
            consume(2 * p, buf0, sem0, p == PAIRS - 1)
            consume(2 * p + 1, buf1, sem1, p == PAIRS - 1)
            return carry

        lax.fori_loop(0, PAIRS, pbody, 0)

        for j in range(BATCHES_PER_WORKER):

            @plsc.parallel_loop(0, 256, 1, unroll=4, carry=jnp.float32(0.0))
            def mx(v, m):
                row = hist[pl.ds(j * HIST1 + v * 16, 16)]
                return jnp.maximum(m, jnp.sum(row))

            res[...] = jnp.full((16,), mx, jnp.float32)
            pltpu.sync_copy(res, out_hbm.at[w * BATCHES_PER_WORKER + j])

    return hist_kernel


_hist_kernel = _make_kernel()


def kernel(inputs):
    x = (
        inputs.transpose(0, 2, 3, 1)
        .reshape(B * ROWS_PER_BATCH // 8, 8, ROW // 128, 128)
        .transpose(0, 2, 1, 3)
        .reshape(-1)
    )
    padded = _hist_kernel(x)
    return padded[:, 0]


# speedup vs baseline: 1.0144x; 1.0144x over previous
import functools

import jax
import jax.numpy as jnp
from jax import lax
from jax.experimental import pallas as pl
from jax.experimental.pallas import tpu as pltpu
from jax.experimental.pallas import tpu_sc as plsc

NUM_CORES = 2
NUM_SUBCORES = 16
NUM_WORKERS = NUM_CORES * NUM_SUBCORES
B = 128
ROW = 1024
ROWS_PER_BATCH = 256
BATCHES_PER_WORKER = B // NUM_WORKERS
CHUNK_ROWS = 32
CHUNK = CHUNK_ROWS * ROW
CHUNKS_PER_BATCH = ROWS_PER_BATCH // CHUNK_ROWS
TOTAL_CHUNKS = BATCHES_PER_WORKER * CHUNKS_PER_BATCH
PAIRS = TOTAL_CHUNKS // 2
HIST1 = 256 * 16
HIST = BATCHES_PER_WORKER * HIST1


def _make_kernel():
    mesh = plsc.VectorSubcoreMesh(
        core_axis_name="c", subcore_axis_name="s", num_cores=NUM_CORES
    )

    @functools.partial(
        pl.kernel,
        mesh=mesh,
        out_type=jax.ShapeDtypeStruct((B,), jnp.float32),
        compiler_params=pltpu.CompilerParams(
            needs_layout_passes=False,
            disable_bounds_checks=True,
            disable_semaphore_checks=True,
            skip_device_barrier=True,
        ),
        scratch_types=[
            pltpu.VMEM((CHUNK,), jnp.int32),
            pltpu.VMEM((CHUNK,), jnp.int32),
            pltpu.VMEM((HIST,), jnp.float32),
            pltpu.VMEM((16,), jnp.float32),
            pltpu.VMEM((16, 16), jnp.float32),
            pltpu.VMEM((64,), jnp.float32),
            plsc.MemoryRef((16, 16), jnp.float32, pltpu.VMEM_SHARED)
            if False
            else pltpu.VMEM_SHARED((16, 16), jnp.float32),
            pltpu.SemaphoreType.DMA,
            pltpu.SemaphoreType.DMA,
        ],
    )
    def hist_kernel(
        x_hbm, out_hbm, buf0, buf1, hist, res, tmp, compact, shared, sem0, sem1
    ):
        c = lax.axis_index("c")
        s = lax.axis_index("s")
        w = c * NUM_SUBCORES + s
        lane = lax.iota(jnp.int32, 16)
        ones = jnp.ones((16,), jnp.float32)
        fzero = jnp.zeros((16,), jnp.float32)
        word0 = w * BATCHES_PER_WORKER * ROWS_PER_BATCH * ROW

        def start(t, buf, sem):
            pltpu.async_copy(
                x_hbm.at[pl.ds(word0 + t * CHUNK, CHUNK)], buf, sem
            )

        def wait(buf, sem):
            pltpu.make_async_copy(
                x_hbm.at[pl.ds(0, CHUNK)], buf, sem
            ).wait()

        start(0, buf0, sem0)
        start(1, buf1, sem1)

        @plsc.parallel_loop(0, HIST, 16, unroll=8)
        def _(i):
            hist[pl.ds(i, 16)] = fzero

        def consume(t, buf, sem, last):
            wait(buf, sem)
            base = lane + (t >> 3) * HIST1

            @plsc.parallel_loop(0, CHUNK, 16, unroll=8)
            def _(i):
                vals = buf[pl.ds(i, 16)]
                idx = (vals << 4) + base
                plsc.addupdate_scatter(hist, [idx], ones)

            @pl.when(jnp.logical_not(last))
            def _():
                start(t + 2, buf, sem)

        def pbody(p, carry):
            consume(2 * p, buf0, sem0, p == PAIRS - 1)
            consume(2 * p + 1, buf1, sem1, p == PAIRS - 1)
            return carry

        lax.fori_loop(0, PAIRS, pbody, 0)

        slot = (s % 4) * 4
        acc = jnp.zeros((16,), jnp.float32)
        for j in range(BATCHES_PER_WORKER):

            @plsc.parallel_loop(0, 256, 1, unroll=4, carry=jnp.float32(0.0))
            def mx(v, m):
                row = hist[pl.ds(j * HIST1 + v * 16, 16)]
                return jnp.maximum(m, jnp.sum(row))

            acc = jnp.where(lane == slot + j, mx, acc)

        res[...] = acc
        pltpu.sync_copy(res, shared.at[s])
        plsc.subcore_barrier()

        @pl.when(s == 0)
        def _():
            pltpu.sync_copy(shared, tmp)
            for k in range(4):
                tot = (
                    tmp[4 * k + 0, pl.ds(0, 16)]
                    + tmp[4 * k + 1, pl.ds(0, 16)]
                    + tmp[4 * k + 2, pl.ds(0, 16)]
                    + tmp[4 * k + 3, pl.ds(0, 16)]
                )
                compact[pl.ds(16 * k, 16)] = tot
            pltpu.sync_copy(compact, out_hbm.at[pl.ds(c * 64, 64)])

    return hist_kernel


_hist_kernel = _make_kernel()


def kernel(inputs):
    x = (
        inputs.transpose(0, 2, 3, 1)
        .reshape(B * ROWS_PER_BATCH // 8, 8, ROW // 128, 128)
        .transpose(0, 2, 1, 3)
        .reshape(-1)
    )
    return _hist_kernel(x)
